# baseline (speedup 1.0000x reference)
import functools

import jax
import jax.numpy as jnp
from jax import lax
from jax.experimental import pallas as pl
from jax.experimental.pallas import tpu as pltpu
from jax.experimental.pallas import tpu_sc as plsc

_NC = 2
_NS = 16
_L = 16
_NW = _NC * _NS

_SUB = 40
_NBUF = 4


def _spmm_sc(src2, dst2, w, feat):
    n_nodes, nwords = feat.shape
    nfeat = 2 * nwords
    n_edges = w.shape[0]
    epw = n_edges // _NW
    nsub = epw // _SUB
    rows_per_tile = n_nodes // _NS

    mesh = plsc.VectorSubcoreMesh(core_axis_name="c", subcore_axis_name="s")

    @functools.partial(
        pl.kernel,
        out_type=jax.ShapeDtypeStruct((_NC * n_nodes, nfeat), jnp.float32),
        mesh=mesh,
        scratch_types=[
            pltpu.VMEM_SHARED((n_nodes, nfeat), jnp.float32),
            pltpu.VMEM((nsub, _SUB), jnp.int32),
            pltpu.VMEM((nsub, _SUB), jnp.int32),
            pltpu.VMEM((epw,), jnp.float32),
        ] + [pltpu.VMEM((_SUB, nwords), jnp.int32) for _ in range(_NBUF)]
          + [pltpu.VMEM((_SUB, nfeat), jnp.float32) for _ in range(2)]
          + [pltpu.SemaphoreType.DMA for _ in range(_NBUF + 2)],
        compiler_params=pltpu.CompilerParams(use_tc_tiling_on_sc=False,
                                             needs_layout_passes=False),
    )
    def spmm_kernel(src_h, dst_h, w_h, feat_h, out_h,
                    acc, sidx, didx, wv, *bufs_and_sems):
        gbufs = bufs_and_sems[:_NBUF]
        sbufs = bufs_and_sems[_NBUF:_NBUF + 2]
        gsems = bufs_and_sems[_NBUF + 2:2 * _NBUF + 2]
        ssems = bufs_and_sems[2 * _NBUF + 2:]
        buf0 = sbufs[0]
        cid = lax.axis_index("c")
        sid = lax.axis_index("s")
        wid = sid * _NC + cid

        pltpu.sync_copy(src_h.at[pl.ds(wid * nsub, nsub)], sidx)
        pltpu.sync_copy(dst_h.at[pl.ds(wid * nsub, nsub)], didx)
        pltpu.sync_copy(w_h.at[pl.ds(wid * epw, epw)], wv)

        r0 = sid * rows_per_tile

        def gissue(t, b):
            pltpu.async_copy(feat_h.at[sidx.at[t]], gbufs[b], gsems[b])

        def swait(sb):
            pltpu.make_async_copy(sbufs[sb], acc.at[didx.at[0]],
                                  ssems[sb]).wait()

        def consume(t, b, sb):
            pltpu.make_async_copy(feat_h.at[sidx.at[t]], gbufs[b],
                                  gsems[b]).wait()
            gbuf = gbufs[b]
            sbuf = sbufs[sb]
            mask = jnp.int32(-65536)

            @plsc.parallel_loop(0, _SUB, unroll=4)
            def _(j):
                wb = plsc.load_gather(
                    wv, [jnp.full((_L,), t * _SUB + j, jnp.int32)])
                for m in range(nwords // _L):
                    v = gbuf[j, pl.ds(m * _L, _L)]
                    lo = plsc.bitcast(lax.shift_left(v, 16), jnp.float32)
                    hi = plsc.bitcast(lax.bitwise_and(v, mask), jnp.float32)
                    sbuf[j, pl.ds(2 * m * _L, _L)] = lo * wb
                    sbuf[j, pl.ds((2 * m + 1) * _L, _L)] = hi * wb

            pltpu.async_copy(sbuf, acc.at[didx.at[t]], ssems[sb], add=True)

        zero = jnp.zeros((_L,), jnp.float32)

        def zrow(i, carry):
            for j in range(nfeat // _L):
                buf0[i, pl.ds(j * _L, _L)] = zero
            return carry

        lax.fori_loop(0, _SUB, zrow, 0)
        zcopies = []
        left = rows_per_tile
        off = 0
        while left > 0:
            step = min(left, _SUB)
            pltpu.async_copy(buf0.at[pl.ds(0, step)],
                             acc.at[pl.ds(r0 + off, step)],
                             ssems[len(zcopies) % 2])
            zcopies.append((step, off))
            off += step
            left -= step
        for zi, (step, off) in enumerate(zcopies):
            pltpu.make_async_copy(buf0.at[pl.ds(0, step)],
                                  acc.at[pl.ds(r0 + off, step)],
                                  ssems[zi % 2]).wait()
        plsc.subcore_barrier()

        gissue(0, 0)
        gissue(1, 1)
        gissue(2, 2)
        consume(0, 0, 0)
        gissue(3, 3)
        consume(1, 1, 1)

        def group(i, carry):
            for k in range(4):
                t = 2 + 4 * i + k
                bp = k
                b = (2 + k) % 4
                swait(k % 2)

                @pl.when(t + 2 < nsub)
                def _():
                    gissue(t + 2, bp)

                consume(t, b, k % 2)
            return carry

        lax.fori_loop(0, (nsub - 2) // 4, group, 0)
        swait(nsub % 2)
        swait((nsub + 1) % 2)
        plsc.subcore_barrier()

        pltpu.sync_copy(acc.at[pl.ds(r0, rows_per_tile)],
                        out_h.at[pl.ds(cid * n_nodes + r0, rows_per_tile)])

    return spmm_kernel


def _mm_fused(ax, W1, b1, W2, block_rows=1000):
    n2, nfeat = ax.shape
    n_nodes = n2 // 2
    nhid2 = W1.shape[1]
    nout = W2.shape[1]

    def body(ax_ref, w1_ref, b1_ref, w2_ref, out_ref):
        s = ax_ref[0] + ax_ref[1]
        h = jnp.dot(s, w1_ref[...], preferred_element_type=jnp.float32)
        h = jnp.maximum(h + b1_ref[...], 0.0)
        t = jnp.dot(h, w2_ref[...], preferred_element_type=jnp.float32)
        out_ref[...] = t.astype(jnp.bfloat16)

    grid = (n_nodes // block_rows,)
    return pl.pallas_call(
        body,
        grid=grid,
        in_specs=[
            pl.BlockSpec((2, block_rows, nfeat), lambda i: (0, i, 0)),
            pl.BlockSpec((nfeat, nhid2), lambda i: (0, 0)),
            pl.BlockSpec((1, nhid2), lambda i: (0, 0)),
            pl.BlockSpec((nhid2, nout), lambda i: (0, 0)),
        ],
        out_specs=pl.BlockSpec((block_rows, nout), lambda i: (i, 0)),
        out_shape=jax.ShapeDtypeStruct((n_nodes, nout), jnp.bfloat16),
    )(ax.reshape(2, n_nodes, nfeat), W1, b1.reshape(1, nhid2), W2)


def _combine(o, b2, block_rows=1000):
    n2, nfeat = o.shape
    n_nodes = n2 // 2

    def body(o_ref, b2_ref, out_ref):
        out_ref[...] = o_ref[0] + o_ref[1] + b2_ref[...]

    return pl.pallas_call(
        body,
        grid=(n_nodes // block_rows,),
        in_specs=[
            pl.BlockSpec((2, block_rows, nfeat), lambda i: (0, i, 0)),
            pl.BlockSpec((1, nfeat), lambda i: (0, 0)),
        ],
        out_specs=pl.BlockSpec((block_rows, nfeat), lambda i: (i, 0)),
        out_shape=jax.ShapeDtypeStruct((n_nodes, nfeat), jnp.float32),
    )(o.reshape(2, n_nodes, nfeat), b2.reshape(1, nfeat))


def _pack_bf16(a, perm):
    n, f = a.shape
    ab = a[:, perm].astype(jnp.bfloat16)
    return jax.lax.bitcast_convert_type(ab.reshape(n, f // 2, 2), jnp.int32)


def kernel(x, edge_index, adj_weight, W1, b1, W2, b2):
    import numpy as np
    src = edge_index[0].astype(jnp.int32).reshape(-1, _SUB)
    dst = edge_index[1].astype(jnp.int32).reshape(-1, _SUB)
    w = adj_weight.astype(jnp.float32)
    nfeat = x.shape[1]

    perm = np.arange(nfeat).reshape(-1, 2, 16).transpose(0, 2, 1).reshape(-1)

    spmm = _spmm_sc(src, dst, w, _pack_bf16(x, perm))
    ax = spmm(src, dst, w, _pack_bf16(x, perm))
    t = _mm_fused(ax, W1, b1, W2[:, perm])
    ti = jax.lax.bitcast_convert_type(
        t.reshape(t.shape[0], t.shape[1] // 2, 2), jnp.int32)
    ot = spmm(src, dst, w, ti)
    return _combine(ot, b2)


# speedup vs baseline: 1.1136x; 1.1136x over previous
import functools

import jax
import jax.numpy as jnp
from jax import lax
from jax.experimental import pallas as pl
from jax.experimental.pallas import tpu as pltpu
from jax.experimental.pallas import tpu_sc as plsc

_NC = 2
_NS = 16
_L = 16
_NW = _NC * _NS

_SUB = 40
_NBUF = 4


def _spmm_sc(src2, dst2, w, feat):
    n_nodes, nfeat = feat.shape
    n_edges = w.shape[0]
    epw = n_edges // _NW
    nsub = epw // _SUB
    rows_per_tile = n_nodes // _NS

    mesh = plsc.VectorSubcoreMesh(core_axis_name="c", subcore_axis_name="s")

    @functools.partial(
        pl.kernel,
        out_type=jax.ShapeDtypeStruct((_NC * n_nodes, nfeat), jnp.float32),
        mesh=mesh,
        scratch_types=[
            pltpu.VMEM_SHARED((n_nodes, nfeat), jnp.float32),
            pltpu.VMEM((nsub, _SUB), jnp.int32),
            pltpu.VMEM((nsub, _SUB), jnp.int32),
            pltpu.VMEM((epw,), jnp.float32),
        ] + [pltpu.VMEM((_SUB, nfeat), jnp.float32) for _ in range(_NBUF)]
          + [pltpu.SemaphoreType.DMA for _ in range(2 * _NBUF)],
        compiler_params=pltpu.CompilerParams(use_tc_tiling_on_sc=False,
                                             needs_layout_passes=False),
    )
    def spmm_kernel(src_h, dst_h, w_h, feat_h, out_h,
                    acc, sidx, didx, wv, *bufs_and_sems):
        bufs = bufs_and_sems[:_NBUF]
        gsems = bufs_and_sems[_NBUF:2 * _NBUF]
        ssems = bufs_and_sems[2 * _NBUF:]
        buf0 = bufs[0]
        cid = lax.axis_index("c")
        sid = lax.axis_index("s")
        wid = sid * _NC + cid

        pltpu.sync_copy(src_h.at[pl.ds(wid * nsub, nsub)], sidx)
        pltpu.sync_copy(dst_h.at[pl.ds(wid * nsub, nsub)], didx)
        pltpu.sync_copy(w_h.at[pl.ds(wid * epw, epw)], wv)

        r0 = sid * rows_per_tile

        def gissue(t, b):
            pltpu.async_copy(feat_h.at[sidx.at[t]], bufs[b], gsems[b])

        def swait(b):
            pltpu.make_async_copy(bufs[b], acc.at[didx.at[0]],
                                  ssems[b]).wait()

        def consume(t, b):
            pltpu.make_async_copy(feat_h.at[sidx.at[t]], bufs[b],
                                  gsems[b]).wait()
            buf = bufs[b]

            @plsc.parallel_loop(0, _SUB, unroll=4)
            def _(j):
                wb = plsc.load_gather(
                    wv, [jnp.full((_L,), t * _SUB + j, jnp.int32)])
                for k in range(nfeat // _L):
                    sl = pl.ds(k * _L, _L)
                    buf[j, sl] = buf[j, sl] * wb

            pltpu.async_copy(buf, acc.at[didx.at[t]], ssems[b], add=True)

        zero = jnp.zeros((_L,), jnp.float32)
        buf0 = bufs[0]

        def zrow(i, carry):
            for j in range(nfeat // _L):
                buf0[i, pl.ds(j * _L, _L)] = zero
            return carry

        lax.fori_loop(0, _SUB, zrow, 0)
        zcopies = []
        left = rows_per_tile
        off = 0
        while left > 0:
            step = min(left, _SUB)
            pltpu.async_copy(buf0.at[pl.ds(0, step)],
                             acc.at[pl.ds(r0 + off, step)],
                             ssems[len(zcopies) % _NBUF])
            zcopies.append((step, off))
            off += step
            left -= step
        for zi, (step, off) in enumerate(zcopies):
            pltpu.make_async_copy(buf0.at[pl.ds(0, step)],
                                  acc.at[pl.ds(r0 + off, step)],
                                  ssems[zi % _NBUF]).wait()
        plsc.subcore_barrier()

        gissue(0, 0)
        gissue(1, 1)
        gissue(2, 2)
        consume(0, 0)
        gissue(3, 3)
        consume(1, 1)

        def group(i, carry):
            for k in range(4):
                t = 2 + 4 * i + k
                bp = k
                b = (2 + k) % 4
                swait(bp)

                @pl.when(t + 2 < nsub)
                def _():
                    gissue(t + 2, bp)

                consume(t, b)
            return carry

        lax.fori_loop(0, (nsub - 2) // 4, group, 0)
        swait((nsub - 2) % _NBUF)
        swait((nsub - 1) % _NBUF)
        plsc.subcore_barrier()

        pltpu.sync_copy(acc.at[pl.ds(r0, rows_per_tile)],
                        out_h.at[pl.ds(cid * n_nodes + r0, rows_per_tile)])

    return spmm_kernel


def _mm_fused(ax, W1, b1, W2, block_rows=1000):
    n2, nfeat = ax.shape
    n_nodes = n2 // 2
    nhid2 = W1.shape[1]
    nout = W2.shape[1]

    def body(ax_ref, w1_ref, b1_ref, w2_ref, out_ref):
        s = ax_ref[0] + ax_ref[1]
        h = jnp.dot(s, w1_ref[...], preferred_element_type=jnp.float32)
        h = jnp.maximum(h + b1_ref[...], 0.0)
        out_ref[...] = jnp.dot(h, w2_ref[...], preferred_element_type=jnp.float32)

    grid = (n_nodes // block_rows,)
    return pl.pallas_call(
        body,
        grid=grid,
        in_specs=[
            pl.BlockSpec((2, block_rows, nfeat), lambda i: (0, i, 0)),
            pl.BlockSpec((nfeat, nhid2), lambda i: (0, 0)),
            pl.BlockSpec((1, nhid2), lambda i: (0, 0)),
            pl.BlockSpec((nhid2, nout), lambda i: (0, 0)),
        ],
        out_specs=pl.BlockSpec((block_rows, nout), lambda i: (i, 0)),
        out_shape=jax.ShapeDtypeStruct((n_nodes, nout), jnp.float32),
    )(ax.reshape(2, n_nodes, nfeat), W1, b1.reshape(1, nhid2), W2)


def _combine(o, b2, block_rows=1000):
    n2, nfeat = o.shape
    n_nodes = n2 // 2

    def body(o_ref, b2_ref, out_ref):
        out_ref[...] = o_ref[0] + o_ref[1] + b2_ref[...]

    return pl.pallas_call(
        body,
        grid=(n_nodes // block_rows,),
        in_specs=[
            pl.BlockSpec((2, block_rows, nfeat), lambda i: (0, i, 0)),
            pl.BlockSpec((1, nfeat), lambda i: (0, 0)),
        ],
        out_specs=pl.BlockSpec((block_rows, nfeat), lambda i: (i, 0)),
        out_shape=jax.ShapeDtypeStruct((n_nodes, nfeat), jnp.float32),
    )(o.reshape(2, n_nodes, nfeat), b2.reshape(1, nfeat))


def kernel(x, edge_index, adj_weight, W1, b1, W2, b2):
    src = edge_index[0].astype(jnp.int32).reshape(-1, _SUB)
    dst = edge_index[1].astype(jnp.int32).reshape(-1, _SUB)
    w = adj_weight.astype(jnp.float32)

    spmm = _spmm_sc(src, dst, w, x)
    ax = spmm(src, dst, w, x)
    t = _mm_fused(ax, W1, b1, W2)
    ot = spmm(src, dst, w, t)
    return _combine(ot, b2)
